# Initial kernel scaffold; baseline (speedup 1.0000x reference)
import jax
import jax.numpy as jnp
from jax.experimental import pallas as pl


def kernel(time, table, max_len):
    raise NotImplementedError("write your pallas kernel here")


# speedup vs baseline: 1.1745x; 1.1745x over previous
import functools

import jax
import jax.numpy as jnp
from jax import lax
from jax.experimental import pallas as pl
from jax.experimental.pallas import tpu as pltpu
from jax.experimental.pallas import tpu_sc as plsc

_NC = 2
_NS = 16
_NW = _NC * _NS

_CHUNK = 1024


def _idx_body(clip, t_ref, idx_ref):
    t = t_ref[...]
    d = jnp.abs(t[:, :, None] - t[:, None, :])
    idx_ref[...] = jnp.minimum(d, clip)


def _pairwise_idx(time, clip):
    b, h = time.shape
    blk = 512
    assert b % blk == 0
    return pl.pallas_call(
        functools.partial(_idx_body, clip),
        grid=(b // blk,),
        in_specs=[pl.BlockSpec((blk, h), lambda i: (i, 0))],
        out_specs=pl.BlockSpec((blk, h, h), lambda i: (i, 0, 0)),
        out_shape=jax.ShapeDtypeStruct((b, h, h), jnp.int32),
    )(time)


def _gather_body(rows_per_w, idx_hbm, table_hbm, out_hbm, idx_v, rows_v, sem):
    wid = lax.axis_index("s") * _NC + lax.axis_index("c")
    base = wid * rows_per_w
    pltpu.sync_copy(idx_hbm.at[pl.ds(base, rows_per_w)], idx_v)

    def chunk(c, carry):
        r0 = c * _CHUNK
        pltpu.async_copy(
            table_hbm.at[idx_v.at[pl.ds(r0, _CHUNK)]], rows_v, sem
        ).wait()
        pltpu.sync_copy(rows_v, out_hbm.at[pl.ds(base + r0, _CHUNK)])
        return carry

    lax.fori_loop(0, rows_per_w // _CHUNK, chunk, 0)


def kernel(time, table, max_len):
    b, h = time.shape
    v, d = table.shape
    clip = v - 1
    idx = _pairwise_idx(time, clip)

    n_rows = b * h * h
    assert n_rows % (_NW * _CHUNK) == 0
    rows_per_w = n_rows // _NW

    idx_flat = idx.reshape(n_rows)

    mesh = plsc.VectorSubcoreMesh(core_axis_name="c", subcore_axis_name="s")
    out = pl.kernel(
        functools.partial(_gather_body, rows_per_w),
        out_type=jax.ShapeDtypeStruct((n_rows, d), jnp.float32),
        mesh=mesh,
        scratch_types=[
            pltpu.VMEM((rows_per_w,), jnp.int32),
            pltpu.VMEM((_CHUNK, d), jnp.float32),
            pltpu.SemaphoreType.DMA,
        ],
        compiler_params=pltpu.CompilerParams(use_tc_tiling_on_sc=False),
    )(idx_flat, table)
    return out.reshape(b, h, h, d)
